# baseline (speedup 1.0000x reference)
import functools

import jax
import jax.numpy as jnp
from jax import lax
from jax.experimental import pallas as pl
from jax.experimental.pallas import tpu as pltpu
from jax.experimental.pallas import tpu_sc as plsc

N = 10000
E = 320000
IN_CH = 128
HID = 128
OUT = 64

NC = 2
NS = 16
TILES = NC * NS
CHUNK = 128
CPT = 80
E_PAD = TILES * CPT * CHUNK
N_PAD = 10240
RPT = N_PAD // NS

BLK = 2000
GRID = N // BLK

_mesh = plsc.VectorSubcoreMesh(
    core_axis_name="c", subcore_axis_name="s", num_cores=NC, num_subcores=NS)


@functools.partial(
    pl.kernel,
    out_type=jax.ShapeDtypeStruct((NC * N_PAD, HID), jnp.float32),
    mesh=_mesh,
    scratch_types=[
        pltpu.VMEM((CPT, CHUNK), jnp.int32),
        pltpu.VMEM((CHUNK, HID), jnp.float32),
        pltpu.VMEM_SHARED((N_PAD, HID), jnp.float32),
    ],
)
def _sc_hist(dst_hbm, out_hbm, dst_v, e0_v, acc):
    cid = lax.axis_index("c")
    sid = lax.axis_index("s")
    wid = cid * NS + sid
    pltpu.sync_copy(dst_hbm.at[wid], dst_v)

    def zfill(i, _):
        for k in range(HID // 16):
            e0_v[i, pl.ds(k * 16, 16)] = jnp.zeros((16,), jnp.float32)
        return _
    lax.fori_loop(0, CHUNK, zfill, None)
    for k in range(RPT // CHUNK):
        pltpu.sync_copy(e0_v, acc.at[pl.ds(sid * RPT + k * CHUNK, CHUNK)])

    e0 = jnp.where(lax.iota(jnp.int32, 16) == 0, 1.0, 0.0)

    def efill(i, _):
        e0_v[i, pl.ds(0, 16)] = e0
        return _
    lax.fori_loop(0, CHUNK, efill, None)
    plsc.subcore_barrier()

    def body(j, _):
        pltpu.sync_copy(e0_v, acc.at[dst_v.at[j]], add=True)
        return _
    lax.fori_loop(0, CPT, body, None)
    plsc.subcore_barrier()

    for k in range(RPT // CHUNK):
        r = sid * RPT + k * CHUNK
        pltpu.sync_copy(acc.at[pl.ds(r, CHUNK)],
                        out_hbm.at[pl.ds(cid * N_PAD + r, CHUNK)])


NBUF = 2
HALVES = 2
HC = CPT // HALVES
HGROUPS = HC // NBUF


@functools.partial(
    pl.kernel,
    out_type=jax.ShapeDtypeStruct((NC * N_PAD, HID), jnp.float32),
    mesh=_mesh,
    scratch_types=[
        pltpu.VMEM((HC, CHUNK), jnp.int32),
        pltpu.VMEM((HC, CHUNK), jnp.int32),
        pltpu.VMEM((CHUNK, HID), jnp.float32),
        pltpu.VMEM((CHUNK, HID), jnp.float32),
        pltpu.VMEM_SHARED((N_PAD, HID), jnp.float32),
        pltpu.SemaphoreType.DMA,
        pltpu.SemaphoreType.DMA,
        pltpu.SemaphoreType.DMA,
        pltpu.SemaphoreType.DMA,
    ],
)
def _sc_scatter(y_hbm, src_hbm, dst_hbm, out_hbm,
                src_v, dst_v, r0, r1, acc, s0, s1, s2, s3):
    bufs = (r0, r1)
    sems = ((s0, s1), (s2, s3))
    HB = CHUNK // 2

    def _gather_halves(op, j, b):
        for p in range(2):
            op(y_hbm.at[src_v.at[j, pl.ds(p * HB, HB)]],
               bufs[b].at[pl.ds(p * HB, HB)], sems[b][p])

    def _issue(j, b):
        _gather_halves(pltpu.async_copy, j, b)

    def _wait(j, b):
        _gather_halves(lambda a, c, s: pltpu.make_async_copy(a, c, s).wait(),
                       j, b)

    cid = lax.axis_index("c")
    sid = lax.axis_index("s")
    wid = cid * NS + sid

    def zero(i, _):
        for k in range(HID // 16):
            r0[i, pl.ds(k * 16, 16)] = jnp.zeros((16,), jnp.float32)
        return _
    lax.fori_loop(0, CHUNK, zero, None)
    for k in range(RPT // CHUNK):
        pltpu.sync_copy(r0, acc.at[pl.ds(sid * RPT + k * CHUNK, CHUNK)])

    for h in range(HALVES):
        pltpu.sync_copy(src_hbm.at[wid, pl.ds(h * HC, HC)], src_v)
        pltpu.sync_copy(dst_hbm.at[wid, pl.ds(h * HC, HC)], dst_v)
        for b in range(NBUF):
            _issue(b, b)
        if h == 0:
            plsc.subcore_barrier()

        def body(i, _):
            for b in range(NBUF):
                j = i * NBUF + b
                _wait(j, b)
                pltpu.sync_copy(bufs[b], acc.at[dst_v.at[j]], add=True)
                _issue(j + NBUF, b)
            return _
        lax.fori_loop(0, HGROUPS - 1, body, None)

        for b in range(NBUF):
            j = (HGROUPS - 1) * NBUF + b
            _wait(j, b)
            pltpu.sync_copy(bufs[b], acc.at[dst_v.at[j]], add=True)
    plsc.subcore_barrier()

    for k in range(RPT // CHUNK):
        r = sid * RPT + k * CHUNK
        pltpu.sync_copy(acc.at[pl.ds(r, CHUNK)],
                        out_hbm.at[pl.ds(cid * N_PAD + r, CHUNK)])


def _dinv(degs_ref):
    d = degs_ref[...]
    return lax.rsqrt(d[:, 0] + d[:, 1] + 1.0)


def _tc_matmul_kern(x_ref, w_ref, o_ref):
    o_ref[...] = jnp.dot(x_ref[...], w_ref[...],
                         preferred_element_type=jnp.float32)


def _tc_scale_kern(z_ref, degs_ref, o_ref):
    o_ref[...] = z_ref[...] * _dinv(degs_ref)[:, None]


def _tc_mid_kern(s_ref, y1_ref, degs_ref, b1_ref, o_ref):
    dinv = _dinv(degs_ref)[:, None]
    agg = dinv * (s_ref[0] + s_ref[1] + y1_ref[...]) + b1_ref[...]
    o_ref[...] = dinv * jnp.maximum(agg, 0.0)


def _tc_out_kern(s_ref, y2_ref, degs_ref, w_ref, b_ref, o_ref):
    dinv = _dinv(degs_ref)[:, None]
    agg = dinv * (s_ref[0] + s_ref[1] + y2_ref[...])
    o_ref[...] = jnp.dot(agg, w_ref[...],
                         preferred_element_type=jnp.float32) + b_ref[...]


def _row_spec(ch):
    return pl.BlockSpec((BLK, ch), lambda i: (i, 0))


_degs_spec = pl.BlockSpec((BLK, NC), lambda i: (i, 0))
_s_spec = pl.BlockSpec((2, BLK, HID), lambda i: (0, i, 0))
_w_spec = pl.BlockSpec((HID, HID), lambda i: (0, 0))
_b_spec = pl.BlockSpec((1, HID), lambda i: (0, 0))


def kernel(x, edge_index, W1, b1, Wmu, bmu, Wlv, blv):
    ei = edge_index.astype(jnp.int32)
    pad = E_PAD - E
    src3 = jnp.concatenate(
        [ei[0], jnp.zeros((pad,), jnp.int32)]).reshape(TILES, CPT, CHUNK)
    dst3 = jnp.concatenate(
        [ei[1], jnp.full((pad,), N, jnp.int32)]).reshape(TILES, CPT, CHUNK)

    degs = _sc_hist(dst3).reshape(NC, N_PAD, HID)[:, :N, 0].T

    z = pl.pallas_call(
        _tc_matmul_kern, grid=(GRID,),
        in_specs=[_row_spec(IN_CH), _w_spec],
        out_specs=_row_spec(HID),
        out_shape=jax.ShapeDtypeStruct((N, HID), jnp.float32),
    )(x, W1)

    y1 = pl.pallas_call(
        _tc_scale_kern, grid=(GRID,),
        in_specs=[_row_spec(HID), _degs_spec],
        out_specs=_row_spec(HID),
        out_shape=jax.ShapeDtypeStruct((N, HID), jnp.float32),
    )(z, degs)

    s1 = _sc_scatter(y1, src3, dst3).reshape(NC, N_PAD, HID)

    y2 = pl.pallas_call(
        _tc_mid_kern, grid=(GRID,),
        in_specs=[_s_spec, _row_spec(HID), _degs_spec, _b_spec],
        out_specs=_row_spec(HID),
        out_shape=jax.ShapeDtypeStruct((N, HID), jnp.float32),
    )(s1, y1, degs, b1.reshape(1, HID))

    s2 = _sc_scatter(y2, src3, dst3).reshape(NC, N_PAD, HID)

    W = jnp.concatenate([Wmu, Wlv], axis=1)
    b = jnp.concatenate([bmu, blv]).reshape(1, 2 * OUT)
    out = pl.pallas_call(
        _tc_out_kern, grid=(GRID,),
        in_specs=[_s_spec, _row_spec(HID), _degs_spec, _w_spec, _b_spec],
        out_specs=_row_spec(HID),
        out_shape=jax.ShapeDtypeStruct((N, 2 * OUT), jnp.float32),
    )(s2, y2, degs, W, b)

    return out[:, :OUT], out[:, OUT:]


# speedup vs baseline: 1.0022x; 1.0022x over previous
import functools

import jax
import jax.numpy as jnp
from jax import lax
from jax.experimental import pallas as pl
from jax.experimental.pallas import tpu as pltpu
from jax.experimental.pallas import tpu_sc as plsc

N = 10000
E = 320000
IN_CH = 128
HID = 128
OUT = 64

NC = 2
NS = 16
TILES = NC * NS
CHUNK = 128
CPT = 80
E_PAD = TILES * CPT * CHUNK
N_PAD = 10240
RPT = N_PAD // NS

BLK = 2000
GRID = N // BLK

_mesh = plsc.VectorSubcoreMesh(
    core_axis_name="c", subcore_axis_name="s", num_cores=NC, num_subcores=NS)


@functools.partial(
    pl.kernel,
    out_type=jax.ShapeDtypeStruct((NC * N_PAD, HID), jnp.float32),
    mesh=_mesh,
    scratch_types=[
        pltpu.VMEM((CPT, CHUNK), jnp.int32),
        pltpu.VMEM((CHUNK, HID), jnp.float32),
        pltpu.VMEM_SHARED((N_PAD, HID), jnp.float32),
    ],
)
def _sc_hist(dst_hbm, out_hbm, dst_v, e0_v, acc):
    cid = lax.axis_index("c")
    sid = lax.axis_index("s")
    wid = cid * NS + sid
    pltpu.sync_copy(dst_hbm.at[wid], dst_v)

    def zfill(i, _):
        for k in range(HID // 16):
            e0_v[i, pl.ds(k * 16, 16)] = jnp.zeros((16,), jnp.float32)
        return _
    lax.fori_loop(0, CHUNK, zfill, None)
    for k in range(RPT // CHUNK):
        pltpu.sync_copy(e0_v, acc.at[pl.ds(sid * RPT + k * CHUNK, CHUNK)])

    e0 = jnp.where(lax.iota(jnp.int32, 16) == 0, 1.0, 0.0)

    def efill(i, _):
        e0_v[i, pl.ds(0, 16)] = e0
        return _
    lax.fori_loop(0, CHUNK, efill, None)
    plsc.subcore_barrier()

    def body(j, _):
        pltpu.sync_copy(e0_v, acc.at[dst_v.at[j]], add=True)
        return _
    lax.fori_loop(0, CPT, body, None)
    plsc.subcore_barrier()

    for k in range(RPT // CHUNK):
        r = sid * RPT + k * CHUNK
        pltpu.sync_copy(acc.at[pl.ds(r, CHUNK)],
                        out_hbm.at[pl.ds(cid * N_PAD + r, CHUNK)])


NBUF = 2
HALVES = 2
HC = CPT // HALVES
HGROUPS = HC // NBUF


@functools.partial(
    pl.kernel,
    out_type=jax.ShapeDtypeStruct((NC * N_PAD, HID), jnp.float32),
    mesh=_mesh,
    scratch_types=[
        pltpu.VMEM((HC, CHUNK), jnp.int32),
        pltpu.VMEM((HC, CHUNK), jnp.int32),
        pltpu.VMEM((CHUNK, HID), jnp.float32),
        pltpu.VMEM((CHUNK, HID), jnp.float32),
        pltpu.VMEM_SHARED((N_PAD, HID), jnp.float32),
        pltpu.SemaphoreType.DMA,
        pltpu.SemaphoreType.DMA,
    ],
)
def _sc_scatter(y_hbm, src_hbm, dst_hbm, out_hbm,
                src_v, dst_v, r0, r1, acc, s0, s1):
    bufs = (r0, r1)
    sems = (s0, s1)

    def _issue(j, b):
        pltpu.async_copy(y_hbm.at[src_v.at[j]], bufs[b], sems[b])

    def _wait(j, b):
        pltpu.make_async_copy(y_hbm.at[src_v.at[j]], bufs[b], sems[b]).wait()

    cid = lax.axis_index("c")
    sid = lax.axis_index("s")
    wid = cid * NS + sid

    def zero(i, _):
        for k in range(HID // 16):
            r0[i, pl.ds(k * 16, 16)] = jnp.zeros((16,), jnp.float32)
        return _
    lax.fori_loop(0, CHUNK, zero, None)
    for k in range(RPT // CHUNK):
        pltpu.sync_copy(r0, acc.at[pl.ds(sid * RPT + k * CHUNK, CHUNK)])

    for h in range(HALVES):
        pltpu.sync_copy(src_hbm.at[wid, pl.ds(h * HC, HC)], src_v)
        pltpu.sync_copy(dst_hbm.at[wid, pl.ds(h * HC, HC)], dst_v)
        for b in range(NBUF):
            _issue(b, b)
        if h == 0:
            plsc.subcore_barrier()

        def body(i, _):
            for b in range(NBUF):
                j = i * NBUF + b
                _wait(j, b)
                pltpu.sync_copy(bufs[b], acc.at[dst_v.at[j]], add=True)
                _issue(j + NBUF, b)
            return _
        lax.fori_loop(0, HGROUPS - 1, body, None)

        for b in range(NBUF):
            j = (HGROUPS - 1) * NBUF + b
            _wait(j, b)
            pltpu.sync_copy(bufs[b], acc.at[dst_v.at[j]], add=True)
    plsc.subcore_barrier()

    for k in range(RPT // CHUNK):
        r = sid * RPT + k * CHUNK
        pltpu.sync_copy(acc.at[pl.ds(r, CHUNK)],
                        out_hbm.at[pl.ds(cid * N_PAD + r, CHUNK)])


def _dinv(degs_ref):
    d = degs_ref[...]
    return lax.rsqrt(d[:, 0] + d[:, 1] + 1.0)


def _tc_matmul_scale_kern(x_ref, w_ref, degs_ref, o_ref):
    z = jnp.dot(x_ref[...], w_ref[...], preferred_element_type=jnp.float32)
    o_ref[...] = z * _dinv(degs_ref)[:, None]


def _tc_mid_kern(s_ref, y1_ref, degs_ref, b1_ref, o_ref):
    dinv = _dinv(degs_ref)[:, None]
    agg = dinv * (s_ref[0] + s_ref[1] + y1_ref[...]) + b1_ref[...]
    o_ref[...] = dinv * jnp.maximum(agg, 0.0)


def _tc_out_kern(s_ref, y2_ref, degs_ref, w_ref, b_ref, o_ref):
    dinv = _dinv(degs_ref)[:, None]
    agg = dinv * (s_ref[0] + s_ref[1] + y2_ref[...])
    o_ref[...] = jnp.dot(agg, w_ref[...],
                         preferred_element_type=jnp.float32) + b_ref[...]


def _row_spec(ch):
    return pl.BlockSpec((BLK, ch), lambda i: (i, 0))


_degs_spec = pl.BlockSpec((BLK, NC), lambda i: (i, 0))
_s_spec = pl.BlockSpec((2, BLK, HID), lambda i: (0, i, 0))
_w_spec = pl.BlockSpec((HID, HID), lambda i: (0, 0))
_b_spec = pl.BlockSpec((1, HID), lambda i: (0, 0))


def kernel(x, edge_index, W1, b1, Wmu, bmu, Wlv, blv):
    ei = edge_index.astype(jnp.int32)
    pad = E_PAD - E
    src3 = jnp.concatenate(
        [ei[0], jnp.zeros((pad,), jnp.int32)]).reshape(TILES, CPT, CHUNK)
    dst3 = jnp.concatenate(
        [ei[1], jnp.full((pad,), N, jnp.int32)]).reshape(TILES, CPT, CHUNK)

    degs = _sc_hist(dst3).reshape(NC, N_PAD, HID)[:, :N, 0].T

    y1 = pl.pallas_call(
        _tc_matmul_scale_kern, grid=(GRID,),
        in_specs=[_row_spec(IN_CH), _w_spec, _degs_spec],
        out_specs=_row_spec(HID),
        out_shape=jax.ShapeDtypeStruct((N, HID), jnp.float32),
    )(x, W1, degs)

    s1 = _sc_scatter(y1, src3, dst3).reshape(NC, N_PAD, HID)

    y2 = pl.pallas_call(
        _tc_mid_kern, grid=(GRID,),
        in_specs=[_s_spec, _row_spec(HID), _degs_spec, _b_spec],
        out_specs=_row_spec(HID),
        out_shape=jax.ShapeDtypeStruct((N, HID), jnp.float32),
    )(s1, y1, degs, b1.reshape(1, HID))

    s2 = _sc_scatter(y2, src3, dst3).reshape(NC, N_PAD, HID)

    W = jnp.concatenate([Wmu, Wlv], axis=1)
    b = jnp.concatenate([bmu, blv]).reshape(1, 2 * OUT)
    out = pl.pallas_call(
        _tc_out_kern, grid=(GRID,),
        in_specs=[_s_spec, _row_spec(HID), _degs_spec, _w_spec, _b_spec],
        out_specs=_row_spec(HID),
        out_shape=jax.ShapeDtypeStruct((N, 2 * OUT), jnp.float32),
    )(s2, y2, degs, W, b)

    return out[:, :OUT], out[:, OUT:]


# speedup vs baseline: 1.0022x; 1.0000x over previous
import functools

import jax
import jax.numpy as jnp
from jax import lax
from jax.experimental import pallas as pl
from jax.experimental.pallas import tpu as pltpu
from jax.experimental.pallas import tpu_sc as plsc

N = 10000
E = 320000
IN_CH = 128
HID = 128
OUT = 64

NC = 2
NS = 16
TILES = NC * NS
CHUNK = 128
CPT = 80
E_PAD = TILES * CPT * CHUNK
N_PAD = 10240
RPT = N_PAD // NS

BLK = 2000
GRID = N // BLK

_mesh = plsc.VectorSubcoreMesh(
    core_axis_name="c", subcore_axis_name="s", num_cores=NC, num_subcores=NS)


@functools.partial(
    pl.kernel,
    out_type=jax.ShapeDtypeStruct((NC * N_PAD, HID), jnp.float32),
    mesh=_mesh,
    scratch_types=[
        pltpu.VMEM((CPT, CHUNK), jnp.int32),
        pltpu.VMEM((CHUNK, HID), jnp.float32),
        pltpu.VMEM_SHARED((N_PAD, HID), jnp.float32),
    ],
)
def _sc_hist(dst_hbm, out_hbm, dst_v, buf, acc):
    cid = lax.axis_index("c")
    sid = lax.axis_index("s")
    wid = cid * NS + sid
    pltpu.sync_copy(dst_hbm.at[wid], dst_v)

    def zero(i, _):
        for k in range(HID // 16):
            buf[i, pl.ds(k * 16, 16)] = jnp.zeros((16,), jnp.float32)
        return _
    lax.fori_loop(0, CHUNK, zero, None)
    for k in range(RPT // CHUNK):
        pltpu.sync_copy(buf, acc.at[pl.ds(sid * RPT + k * CHUNK, CHUNK)])

    def fill(i, _):
        for k in range(HID // 16):
            buf[i, pl.ds(k * 16, 16)] = jnp.full((16,), 1.0, jnp.float32)
        return _
    lax.fori_loop(0, CHUNK, fill, None)
    plsc.subcore_barrier()

    def body(j, _):
        pltpu.sync_copy(buf, acc.at[dst_v.at[j]], add=True)
        return _
    lax.fori_loop(0, CPT, body, None)
    plsc.subcore_barrier()

    for k in range(RPT // CHUNK):
        r = sid * RPT + k * CHUNK
        pltpu.sync_copy(acc.at[pl.ds(r, CHUNK)],
                        out_hbm.at[pl.ds(cid * N_PAD + r, CHUNK)])


NBUF = 2
HALVES = 2
HC = CPT // HALVES
HGROUPS = HC // NBUF


@functools.partial(
    pl.kernel,
    out_type=jax.ShapeDtypeStruct((NC * N_PAD, HID), jnp.float32),
    mesh=_mesh,
    scratch_types=[
        pltpu.VMEM((HC, CHUNK), jnp.int32),
        pltpu.VMEM((HC, CHUNK), jnp.int32),
        pltpu.VMEM((CHUNK, HID), jnp.float32),
        pltpu.VMEM((CHUNK, HID), jnp.float32),
        pltpu.VMEM_SHARED((N_PAD, HID), jnp.float32),
        pltpu.SemaphoreType.DMA,
        pltpu.SemaphoreType.DMA,
    ],
)
def _sc_scatter(y_hbm, src_hbm, dst_hbm, out_hbm,
                src_v, dst_v, r0, r1, acc, s0, s1):
    bufs = (r0, r1)
    sems = (s0, s1)

    def _issue(j, b):
        pltpu.async_copy(y_hbm.at[src_v.at[j]], bufs[b], sems[b])

    def _wait(j, b):
        pltpu.make_async_copy(y_hbm.at[src_v.at[j]], bufs[b], sems[b]).wait()

    cid = lax.axis_index("c")
    sid = lax.axis_index("s")
    wid = cid * NS + sid

    def zero(i, _):
        for k in range(HID // 16):
            r0[i, pl.ds(k * 16, 16)] = jnp.zeros((16,), jnp.float32)
        return _
    lax.fori_loop(0, CHUNK, zero, None)
    for k in range(RPT // CHUNK):
        pltpu.sync_copy(r0, acc.at[pl.ds(sid * RPT + k * CHUNK, CHUNK)])

    for h in range(HALVES):
        pltpu.sync_copy(src_hbm.at[wid, pl.ds(h * HC, HC)], src_v)
        pltpu.sync_copy(dst_hbm.at[wid, pl.ds(h * HC, HC)], dst_v)
        for b in range(NBUF):
            _issue(b, b)
        if h == 0:
            plsc.subcore_barrier()

        def body(i, _):
            for b in range(NBUF):
                j = i * NBUF + b
                _wait(j, b)
                pltpu.sync_copy(bufs[b], acc.at[dst_v.at[j]], add=True)
                _issue(j + NBUF, b)
            return _
        lax.fori_loop(0, HGROUPS - 1, body, None)

        for b in range(NBUF):
            j = (HGROUPS - 1) * NBUF + b
            _wait(j, b)
            pltpu.sync_copy(bufs[b], acc.at[dst_v.at[j]], add=True)
    plsc.subcore_barrier()

    for k in range(RPT // CHUNK):
        r = sid * RPT + k * CHUNK
        pltpu.sync_copy(acc.at[pl.ds(r, CHUNK)],
                        out_hbm.at[pl.ds(cid * N_PAD + r, CHUNK)])


def _dinv(degs_ref):
    d = degs_ref[...]
    return lax.rsqrt(d[:, 0] + d[:, 1] + 1.0)


def _tc_matmul_scale_kern(x_ref, w_ref, degs_ref, o_ref):
    z = jnp.dot(x_ref[...], w_ref[...], preferred_element_type=jnp.float32)
    o_ref[...] = z * _dinv(degs_ref)[:, None]


def _tc_mid_kern(s_ref, y1_ref, degs_ref, b1_ref, o_ref):
    dinv = _dinv(degs_ref)[:, None]
    agg = dinv * (s_ref[0] + s_ref[1] + y1_ref[...]) + b1_ref[...]
    o_ref[...] = dinv * jnp.maximum(agg, 0.0)


def _tc_out_kern(s_ref, y2_ref, degs_ref, w_ref, b_ref, o_ref):
    dinv = _dinv(degs_ref)[:, None]
    agg = dinv * (s_ref[0] + s_ref[1] + y2_ref[...])
    o_ref[...] = jnp.dot(agg, w_ref[...],
                         preferred_element_type=jnp.float32) + b_ref[...]


def _row_spec(ch):
    return pl.BlockSpec((BLK, ch), lambda i: (i, 0))


_degs_spec = pl.BlockSpec((BLK, NC), lambda i: (i, 0))
_s_spec = pl.BlockSpec((2, BLK, HID), lambda i: (0, i, 0))
_w_spec = pl.BlockSpec((HID, HID), lambda i: (0, 0))
_b_spec = pl.BlockSpec((1, HID), lambda i: (0, 0))


def kernel(x, edge_index, W1, b1, Wmu, bmu, Wlv, blv):
    ei = edge_index.astype(jnp.int32)
    pad = E_PAD - E
    src3 = jnp.concatenate(
        [ei[0], jnp.zeros((pad,), jnp.int32)]).reshape(TILES, CPT, CHUNK)
    dst3 = jnp.concatenate(
        [ei[1], jnp.full((pad,), N, jnp.int32)]).reshape(TILES, CPT, CHUNK)

    degs = _sc_hist(dst3).reshape(NC, N_PAD, HID)[:, :N, 0].T

    y1 = pl.pallas_call(
        _tc_matmul_scale_kern, grid=(GRID,),
        in_specs=[_row_spec(IN_CH), _w_spec, _degs_spec],
        out_specs=_row_spec(HID),
        out_shape=jax.ShapeDtypeStruct((N, HID), jnp.float32),
    )(x, W1, degs)

    s1 = _sc_scatter(y1, src3, dst3).reshape(NC, N_PAD, HID)

    y2 = pl.pallas_call(
        _tc_mid_kern, grid=(GRID,),
        in_specs=[_s_spec, _row_spec(HID), _degs_spec, _b_spec],
        out_specs=_row_spec(HID),
        out_shape=jax.ShapeDtypeStruct((N, HID), jnp.float32),
    )(s1, y1, degs, b1.reshape(1, HID))

    s2 = _sc_scatter(y2, src3, dst3).reshape(NC, N_PAD, HID)

    W = jnp.concatenate([Wmu, Wlv], axis=1)
    b = jnp.concatenate([bmu, blv]).reshape(1, 2 * OUT)
    out = pl.pallas_call(
        _tc_out_kern, grid=(GRID,),
        in_specs=[_s_spec, _row_spec(HID), _degs_spec, _w_spec, _b_spec],
        out_specs=_row_spec(HID),
        out_shape=jax.ShapeDtypeStruct((N, 2 * OUT), jnp.float32),
    )(s2, y2, degs, W, b)

    return out[:, :OUT], out[:, OUT:]
